# Initial kernel scaffold; baseline (speedup 1.0000x reference)
import jax
import jax.numpy as jnp
from jax.experimental import pallas as pl


def kernel(x, params):
    raise NotImplementedError("write your pallas kernel here")


# speedup vs baseline: 9.6458x; 9.6458x over previous
import functools

import jax
import jax.numpy as jnp
from jax import lax
from jax.experimental import pallas as pl
from jax.experimental.pallas import tpu as pltpu
from jax.experimental.pallas import tpu_sc as plsc

B = 4
N = 1024
BN = B * N
K_S = 20
K_L = 40
RB = 256
RBE = 128
NEG = -3.0e38
_INTERPRET = False


def _lrelu(x):
    return jnp.where(x >= 0, x, 0.2 * x)


def _bn_args(p):
    return (p['mean'][None, :], jnp.sqrt(p['var'] + 1e-5)[None, :],
            p['gamma'][None, :], p['beta'][None, :])


def _prep_body(xt_ref, xT_ref, i40_ref):
    xt = xt_ref[0]
    xT = xT_ref[0]
    inner = jnp.dot(xt, xT, preferred_element_type=jnp.float32)
    xxb = jnp.sum(xt * xt, axis=1, keepdims=True)
    xxf = jnp.sum(xT * xT, axis=0, keepdims=True)
    val = -xxb + 2.0 * inner - xxf

    lane = lax.broadcasted_iota(jnp.int32, (RB, N), 1)
    i40l = lax.broadcasted_iota(jnp.int32, (RB, K_L), 1)
    acc = jnp.zeros((RB, K_L), jnp.int32)
    for i in range(K_L):
        m = jnp.max(val, axis=1, keepdims=True)
        am = jnp.min(jnp.where(val == m, lane, N), axis=1)
        acc = jnp.where(i40l == i, am[:, None], acc)
        val = jnp.where(lane == am[:, None], NEG, val)

    i40_ref[0] = acc + pl.program_id(0) * N


def _prep(xt, xT):
    c = xt.shape[-1]
    return pl.pallas_call(
        _prep_body,
        grid=(B, N // RB),
        in_specs=[
            pl.BlockSpec((1, RB, c), lambda b, r: (b, r, 0)),
            pl.BlockSpec((1, c, N), lambda b, r: (b, 0, 0)),
        ],
        out_specs=pl.BlockSpec((1, RB, K_L), lambda b, r: (b, r, 0)),
        out_shape=jax.ShapeDtypeStruct((B, N, K_L), jnp.int32),
        interpret=_INTERPRET,
    )(xt, xT)


def _make_gather(c, p_chunk):
    ppw = BN // 32
    nchunks = ppw // p_chunk
    seg = 80
    ns40 = (p_chunk * K_L) // seg
    mesh = plsc.VectorSubcoreMesh(core_axis_name="c", subcore_axis_name="s")

    @functools.partial(
        pl.kernel, mesh=mesh,
        compiler_params=pltpu.CompilerParams(use_tc_tiling_on_sc=False),
        out_type=jax.ShapeDtypeStruct((BN * K_L, c), jnp.float32),
        scratch_types=[
            pltpu.VMEM((p_chunk * K_L,), jnp.int32),
            pltpu.VMEM((p_chunk * K_L, c), jnp.float32),
            pltpu.SemaphoreType.DMA,
        ],
    )
    def k(i40_hbm, x_hbm, xg_hbm, i40v, rows, sem):
        wid = lax.axis_index("s") * 2 + lax.axis_index("c")

        def chunk(t, carry):
            base = wid * ppw + t * p_chunk
            pltpu.sync_copy(i40_hbm.at[pl.ds(base * K_L, p_chunk * K_L)], i40v)
            for s in range(ns40):
                pltpu.async_copy(
                    x_hbm.at[i40v.at[pl.ds(s * seg, seg)]],
                    rows.at[pl.ds(s * seg, seg)], sem)
            for s in range(ns40):
                pltpu.make_async_copy(
                    x_hbm.at[i40v.at[pl.ds(s * seg, seg)]],
                    rows.at[pl.ds(s * seg, seg)], sem).wait()
            pltpu.sync_copy(rows,
                            xg_hbm.at[pl.ds(base * K_L, p_chunk * K_L)])
            return carry

        lax.fori_loop(0, nchunks, chunk, 0)

    return k


def _gather(i40, x_flat):
    c = x_flat.shape[-1]
    k = _make_gather(c, 16)
    return k(i40.reshape(-1), x_flat)


def _edge_body(c, co, xt_ref, xg_ref, wst_ref, wlt_ref, wft_ref,
               ms_ref, qs_ref, gs_ref, bs_ref,
               ml_ref, ql_ref, gl_ref, bl_ref,
               mf_ref, qf_ref, gf_ref, bf_ref, out_ref):
    xt = xt_ref[...][:, :c]
    xg = xg_ref[...][:, :, :c]
    d = xg - xt[:, None, :]
    xb = jnp.broadcast_to(xt[:, None, :], d.shape)
    feat = jnp.concatenate([d, xb], axis=-1)

    def branch(fk, k, wt, m, q, g, bb):
        y = jnp.dot(fk.reshape(RBE * k, 2 * c), wt,
                    preferred_element_type=jnp.float32)
        y = y.reshape(RBE, k, co)
        y = (y - m[None, :, :]) / q[None, :, :] * g[None, :, :] + bb[None, :, :]
        return jnp.max(_lrelu(y), axis=1)

    fs = branch(feat[:, :K_S, :], K_S, wst_ref[...], ms_ref[...], qs_ref[...],
                gs_ref[...], bs_ref[...])
    fl = branch(feat, K_L, wlt_ref[...], ml_ref[...], ql_ref[...],
                gl_ref[...], bl_ref[...])
    cat = jnp.concatenate([fs, fl], axis=-1)
    y2 = jnp.dot(cat, wft_ref[...], preferred_element_type=jnp.float32)
    y2 = (y2 - mf_ref[...]) / qf_ref[...] * gf_ref[...] + bf_ref[...]
    out_ref[...] = _lrelu(y2)


def _edge(xt_flat, xg, c, co, wst, wlt, wft, bns, bnl, bnf):
    cp = xt_flat.shape[-1]
    full = lambda shape: pl.BlockSpec(shape, lambda i: (0,) * len(shape))
    return pl.pallas_call(
        functools.partial(_edge_body, c, co),
        grid=(BN // RBE,),
        in_specs=[
            pl.BlockSpec((RBE, cp), lambda i: (i, 0)),
            pl.BlockSpec((RBE, K_L, cp), lambda i: (i, 0, 0)),
            full((2 * c, co)), full((2 * c, co)), full((2 * co, co)),
            full((1, co)), full((1, co)), full((1, co)), full((1, co)),
            full((1, co)), full((1, co)), full((1, co)), full((1, co)),
            full((1, co)), full((1, co)), full((1, co)), full((1, co)),
        ],
        out_specs=pl.BlockSpec((RBE, co), lambda i: (i, 0)),
        out_shape=jax.ShapeDtypeStruct((BN, co), jnp.float32),
        interpret=_INTERPRET,
    )(xt_flat, xg, wst, wlt, wft, *bns, *bnl, *bnf)


def _head_body(cat_ref, w1t_ref, m1_ref, q1_ref, g1_ref, b1_ref,
               w2t_ref, m2_ref, q2_ref, g2_ref, b2_ref, w3t_ref, b3_ref,
               out_ref):
    h = jnp.dot(cat_ref[...], w1t_ref[...], preferred_element_type=jnp.float32)
    h = _lrelu((h - m1_ref[...]) / q1_ref[...] * g1_ref[...] + b1_ref[...])
    h = jnp.dot(h, w2t_ref[...], preferred_element_type=jnp.float32)
    h = _lrelu((h - m2_ref[...]) / q2_ref[...] * g2_ref[...] + b2_ref[...])
    out_ref[...] = (jnp.dot(h, w3t_ref[...], preferred_element_type=jnp.float32)
                    + b3_ref[...])


def _head(cat, w1t, bn1, w2t, bn2, w3t, b3):
    full = lambda shape: pl.BlockSpec(shape, lambda i: (0,) * len(shape))
    return pl.pallas_call(
        _head_body,
        grid=(BN // RB,),
        in_specs=[pl.BlockSpec((RB, 512), lambda i: (i, 0)),
                  full((512, 256)),
                  full((1, 256)), full((1, 256)), full((1, 256)),
                  full((1, 256)),
                  full((256, 128)),
                  full((1, 128)), full((1, 128)), full((1, 128)),
                  full((1, 128)),
                  full((128, 13)), full((1, 13))],
        out_specs=pl.BlockSpec((RB, 13), lambda i: (i, 0)),
        out_shape=jax.ShapeDtypeStruct((BN, 13), jnp.float32),
        interpret=_INTERPRET,
    )(cat, w1t, *bn1, w2t, *bn2, w3t, b3)


def _layer(xt, p, cout):
    c = xt.shape[-1]
    xT = jnp.transpose(xt, (0, 2, 1))
    i40 = _prep(xt, xT)

    cp = max(16, c)
    xt_p = xt if cp == c else jnp.pad(xt, ((0, 0), (0, 0), (0, cp - c)))
    xg = _gather(i40, xt_p.reshape(BN, cp))
    xg = xg.reshape(BN, K_L, cp)

    out = _edge(xt_p.reshape(BN, cp), xg, c, cout,
                p['ws'].T, p['wl'].T, p['wf'].T,
                _bn_args(p['bns']), _bn_args(p['bnl']), _bn_args(p['bnf']))
    return out.reshape(B, N, cout)


def kernel(x, params):
    h1 = _layer(x, params['ec1'], 64)
    h2 = _layer(h1, params['ec2'], 64)
    h3 = _layer(h2, params['ec3'], 128)
    h4 = _layer(h3, params['ec4'], 256)

    cat = jnp.concatenate([h.reshape(BN, -1) for h in (h1, h2, h3, h4)],
                          axis=-1)
    out = _head(cat, params['w1'].T, _bn_args(params['bn1']),
                params['w2'].T, _bn_args(params['bn2']),
                params['w3'].T, params['b3'][None, :])
    return out.reshape(B, N, 13)


# speedup vs baseline: 11.9823x; 1.2422x over previous
import functools

import jax
import jax.numpy as jnp
from jax import lax
from jax.experimental import pallas as pl
from jax.experimental.pallas import tpu as pltpu
from jax.experimental.pallas import tpu_sc as plsc

B = 4
N = 1024
BN = B * N
K_S = 20
K_L = 40
RB = 256
RBE = 128
NEG = -3.0e38
_INTERPRET = False


def _lrelu(x):
    return jnp.where(x >= 0, x, 0.2 * x)


def _bn_args(p):
    return (p['mean'][None, :], jnp.sqrt(p['var'] + 1e-5)[None, :],
            p['gamma'][None, :], p['beta'][None, :])


def _prep_body(xt_ref, xT_ref, i40_ref):
    xt = xt_ref[0]
    xT = xT_ref[0]
    inner = jnp.dot(xt, xT, preferred_element_type=jnp.float32)
    xxb = jnp.sum(xt * xt, axis=1, keepdims=True)
    xxf = jnp.sum(xT * xT, axis=0, keepdims=True)
    val = -xxb + 2.0 * inner - xxf

    lane = lax.broadcasted_iota(jnp.int32, (RB, N), 1)
    i40l = lax.broadcasted_iota(jnp.int32, (RB, K_L), 1)
    acc = jnp.zeros((RB, K_L), jnp.int32)
    for i in range(K_L):
        am = jnp.argmax(val, axis=1).astype(jnp.int32)
        acc = jnp.where(i40l == i, am[:, None], acc)
        val = jnp.where(lane == am[:, None], NEG, val)

    i40_ref[0] = acc + pl.program_id(0) * N


def _prep(xt, xT):
    c = xt.shape[-1]
    return pl.pallas_call(
        _prep_body,
        grid=(B, N // RB),
        in_specs=[
            pl.BlockSpec((1, RB, c), lambda b, r: (b, r, 0)),
            pl.BlockSpec((1, c, N), lambda b, r: (b, 0, 0)),
        ],
        out_specs=pl.BlockSpec((1, RB, K_L), lambda b, r: (b, r, 0)),
        out_shape=jax.ShapeDtypeStruct((B, N, K_L), jnp.int32),
        interpret=_INTERPRET,
    )(xt, xT)


def _make_gather(c, p_chunk):
    ppw = BN // 32
    nchunks = ppw // p_chunk
    seg = 80
    ns40 = (p_chunk * K_L) // seg
    mesh = plsc.VectorSubcoreMesh(core_axis_name="c", subcore_axis_name="s")

    @functools.partial(
        pl.kernel, mesh=mesh,
        compiler_params=pltpu.CompilerParams(use_tc_tiling_on_sc=False),
        out_type=jax.ShapeDtypeStruct((BN * K_L, c), jnp.float32),
        scratch_types=[
            pltpu.VMEM((p_chunk * K_L,), jnp.int32),
            pltpu.VMEM((p_chunk * K_L, c), jnp.float32),
            pltpu.SemaphoreType.DMA,
        ],
    )
    def k(i40_hbm, x_hbm, xg_hbm, i40v, rows, sem):
        wid = lax.axis_index("s") * 2 + lax.axis_index("c")

        def chunk(t, carry):
            base = wid * ppw + t * p_chunk
            pltpu.sync_copy(i40_hbm.at[pl.ds(base * K_L, p_chunk * K_L)], i40v)
            for s in range(ns40):
                pltpu.async_copy(
                    x_hbm.at[i40v.at[pl.ds(s * seg, seg)]],
                    rows.at[pl.ds(s * seg, seg)], sem)
            for s in range(ns40):
                pltpu.make_async_copy(
                    x_hbm.at[i40v.at[pl.ds(s * seg, seg)]],
                    rows.at[pl.ds(s * seg, seg)], sem).wait()
            pltpu.sync_copy(rows,
                            xg_hbm.at[pl.ds(base * K_L, p_chunk * K_L)])
            return carry

        lax.fori_loop(0, nchunks, chunk, 0)

    return k


def _gather(i40, x_flat):
    c = x_flat.shape[-1]
    k = _make_gather(c, 16)
    return k(i40.reshape(-1), x_flat)


def _edge_body(c, co, xt_ref, xg_ref, wst_ref, wlt_ref, wft_ref,
               ms_ref, qs_ref, gs_ref, bs_ref,
               ml_ref, ql_ref, gl_ref, bl_ref,
               mf_ref, qf_ref, gf_ref, bf_ref, out_ref):
    xt = xt_ref[...][:, :c]
    xg = xg_ref[...][:, :, :c]
    d = xg - xt[:, None, :]
    xb = jnp.broadcast_to(xt[:, None, :], d.shape)
    feat = jnp.concatenate([d, xb], axis=-1)

    def branch(fk, k, wt, m, q, g, bb):
        y = jnp.dot(fk.reshape(RBE * k, 2 * c), wt,
                    preferred_element_type=jnp.float32)
        y = y.reshape(RBE, k, co)
        y = (y - m[None, :, :]) / q[None, :, :] * g[None, :, :] + bb[None, :, :]
        return jnp.max(_lrelu(y), axis=1)

    fs = branch(feat[:, :K_S, :], K_S, wst_ref[...], ms_ref[...], qs_ref[...],
                gs_ref[...], bs_ref[...])
    fl = branch(feat, K_L, wlt_ref[...], ml_ref[...], ql_ref[...],
                gl_ref[...], bl_ref[...])
    cat = jnp.concatenate([fs, fl], axis=-1)
    y2 = jnp.dot(cat, wft_ref[...], preferred_element_type=jnp.float32)
    y2 = (y2 - mf_ref[...]) / qf_ref[...] * gf_ref[...] + bf_ref[...]
    out_ref[...] = _lrelu(y2)


def _edge(xt_flat, xg, c, co, wst, wlt, wft, bns, bnl, bnf):
    cp = xt_flat.shape[-1]
    full = lambda shape: pl.BlockSpec(shape, lambda i: (0,) * len(shape))
    return pl.pallas_call(
        functools.partial(_edge_body, c, co),
        grid=(BN // RBE,),
        in_specs=[
            pl.BlockSpec((RBE, cp), lambda i: (i, 0)),
            pl.BlockSpec((RBE, K_L, cp), lambda i: (i, 0, 0)),
            full((2 * c, co)), full((2 * c, co)), full((2 * co, co)),
            full((1, co)), full((1, co)), full((1, co)), full((1, co)),
            full((1, co)), full((1, co)), full((1, co)), full((1, co)),
            full((1, co)), full((1, co)), full((1, co)), full((1, co)),
        ],
        out_specs=pl.BlockSpec((RBE, co), lambda i: (i, 0)),
        out_shape=jax.ShapeDtypeStruct((BN, co), jnp.float32),
        interpret=_INTERPRET,
    )(xt_flat, xg, wst, wlt, wft, *bns, *bnl, *bnf)


def _head_body(cat_ref, w1t_ref, m1_ref, q1_ref, g1_ref, b1_ref,
               w2t_ref, m2_ref, q2_ref, g2_ref, b2_ref, w3t_ref, b3_ref,
               out_ref):
    h = jnp.dot(cat_ref[...], w1t_ref[...], preferred_element_type=jnp.float32)
    h = _lrelu((h - m1_ref[...]) / q1_ref[...] * g1_ref[...] + b1_ref[...])
    h = jnp.dot(h, w2t_ref[...], preferred_element_type=jnp.float32)
    h = _lrelu((h - m2_ref[...]) / q2_ref[...] * g2_ref[...] + b2_ref[...])
    out_ref[...] = (jnp.dot(h, w3t_ref[...], preferred_element_type=jnp.float32)
                    + b3_ref[...])


def _head(cat, w1t, bn1, w2t, bn2, w3t, b3):
    full = lambda shape: pl.BlockSpec(shape, lambda i: (0,) * len(shape))
    return pl.pallas_call(
        _head_body,
        grid=(BN // RB,),
        in_specs=[pl.BlockSpec((RB, 512), lambda i: (i, 0)),
                  full((512, 256)),
                  full((1, 256)), full((1, 256)), full((1, 256)),
                  full((1, 256)),
                  full((256, 128)),
                  full((1, 128)), full((1, 128)), full((1, 128)),
                  full((1, 128)),
                  full((128, 13)), full((1, 13))],
        out_specs=pl.BlockSpec((RB, 13), lambda i: (i, 0)),
        out_shape=jax.ShapeDtypeStruct((BN, 13), jnp.float32),
        interpret=_INTERPRET,
    )(cat, w1t, *bn1, w2t, *bn2, w3t, b3)


def _layer(xt, p, cout):
    c = xt.shape[-1]
    xT = jnp.transpose(xt, (0, 2, 1))
    i40 = _prep(xt, xT)

    cp = max(16, c)
    xt_p = xt if cp == c else jnp.pad(xt, ((0, 0), (0, 0), (0, cp - c)))
    xg = _gather(i40, xt_p.reshape(BN, cp))
    xg = xg.reshape(BN, K_L, cp)

    out = _edge(xt_p.reshape(BN, cp), xg, c, cout,
                p['ws'].T, p['wl'].T, p['wf'].T,
                _bn_args(p['bns']), _bn_args(p['bnl']), _bn_args(p['bnf']))
    return out.reshape(B, N, cout)


def kernel(x, params):
    h1 = _layer(x, params['ec1'], 64)
    h2 = _layer(h1, params['ec2'], 64)
    h3 = _layer(h2, params['ec3'], 128)
    h4 = _layer(h3, params['ec4'], 256)

    cat = jnp.concatenate([h.reshape(BN, -1) for h in (h1, h2, h3, h4)],
                          axis=-1)
    out = _head(cat, params['w1'].T, _bn_args(params['bn1']),
                params['w2'].T, _bn_args(params['bn2']),
                params['w3'].T, params['b3'][None, :])
    return out.reshape(B, N, 13)


# speedup vs baseline: 13.2817x; 1.1084x over previous
import functools

import jax
import jax.numpy as jnp
from jax import lax
from jax.experimental import pallas as pl
from jax.experimental.pallas import tpu as pltpu
from jax.experimental.pallas import tpu_sc as plsc

B = 4
N = 1024
BN = B * N
K_S = 20
K_L = 40
RB = 256
RBE = 128
NEG = -3.0e38
_INTERPRET = False


def _lrelu(x):
    return jnp.where(x >= 0, x, 0.2 * x)


def _bn_args(p):
    return (p['mean'][None, :], jnp.sqrt(p['var'] + 1e-5)[None, :],
            p['gamma'][None, :], p['beta'][None, :])


def _prep_body(boff, xt_ref, xT_ref, i40_ref):
    xt = xt_ref[...]
    xT = xT_ref[...]
    inner = jnp.dot(xt, xT, preferred_element_type=jnp.float32)
    xxb = jnp.sum(xt * xt, axis=1, keepdims=True)
    xxf = jnp.sum(xT * xT, axis=0, keepdims=True)
    val = -xxb + 2.0 * inner - xxf

    lane = lax.broadcasted_iota(jnp.int32, (RB, N), 1)
    i40l = lax.broadcasted_iota(jnp.int32, (RB, K_L), 1)
    acc = jnp.zeros((RB, K_L), jnp.int32)
    for i in range(K_L):
        am = jnp.argmax(val, axis=1).astype(jnp.int32)
        acc = jnp.where(i40l == i, am[:, None], acc)
        val = jnp.where(lane == am[:, None], NEG, val)

    i40_ref[...] = acc + boff


def _prep(xt2, xT2, boff):
    c = xt2.shape[-1]
    return pl.pallas_call(
        functools.partial(_prep_body, boff),
        grid=(N // RB,),
        in_specs=[
            pl.BlockSpec((RB, c), lambda r: (r, 0)),
            pl.BlockSpec((c, N), lambda r: (0, 0)),
        ],
        out_specs=pl.BlockSpec((RB, K_L), lambda r: (r, 0)),
        out_shape=jax.ShapeDtypeStruct((N, K_L), jnp.int32),
        interpret=_INTERPRET,
    )(xt2, xT2)


def _make_gather(c, p_chunk):
    ppw = N // 32
    nchunks = ppw // p_chunk
    seg = 80
    ns40 = (p_chunk * K_L) // seg
    mesh = plsc.VectorSubcoreMesh(core_axis_name="c", subcore_axis_name="s")

    @functools.partial(
        pl.kernel, mesh=mesh,
        compiler_params=pltpu.CompilerParams(use_tc_tiling_on_sc=False),
        out_type=jax.ShapeDtypeStruct((N * K_L, c), jnp.float32),
        scratch_types=[
            pltpu.VMEM((p_chunk * K_L,), jnp.int32),
            pltpu.VMEM((p_chunk * K_L, c), jnp.float32),
            pltpu.SemaphoreType.DMA,
        ],
    )
    def k(i40_hbm, x_hbm, xg_hbm, i40v, rows, sem):
        wid = lax.axis_index("s") * 2 + lax.axis_index("c")

        def chunk(t, carry):
            base = wid * ppw + t * p_chunk
            pltpu.sync_copy(i40_hbm.at[pl.ds(base * K_L, p_chunk * K_L)], i40v)
            for s in range(ns40):
                pltpu.async_copy(
                    x_hbm.at[i40v.at[pl.ds(s * seg, seg)]],
                    rows.at[pl.ds(s * seg, seg)], sem)
            for s in range(ns40):
                pltpu.make_async_copy(
                    x_hbm.at[i40v.at[pl.ds(s * seg, seg)]],
                    rows.at[pl.ds(s * seg, seg)], sem).wait()
            pltpu.sync_copy(rows,
                            xg_hbm.at[pl.ds(base * K_L, p_chunk * K_L)])
            return carry

        lax.fori_loop(0, nchunks, chunk, 0)

    return k


def _gather(i40, x_flat):
    c = x_flat.shape[-1]
    k = _make_gather(c, 16)
    return k(i40.reshape(-1), x_flat)


def _edge_body(c, co, xt_ref, xg_ref, wst_ref, wlt_ref, wft_ref,
               ms_ref, qs_ref, gs_ref, bs_ref,
               ml_ref, ql_ref, gl_ref, bl_ref,
               mf_ref, qf_ref, gf_ref, bf_ref, out_ref):
    xt = xt_ref[...][:, :c]
    xg = xg_ref[...][:, :, :c]
    d = xg - xt[:, None, :]
    xb = jnp.broadcast_to(xt[:, None, :], d.shape)
    feat = jnp.concatenate([d, xb], axis=-1)

    def branch(fk, k, wt, m, q, g, bb):
        y = jnp.dot(fk.reshape(RBE * k, 2 * c), wt,
                    preferred_element_type=jnp.float32)
        y = y.reshape(RBE, k, co)
        mx = jnp.max(y, axis=1)
        mn = jnp.min(y, axis=1)
        ybn = jnp.where(g >= 0, mx, mn)
        return _lrelu((ybn - m) / q * g + bb)

    fs = branch(feat[:, :K_S, :], K_S, wst_ref[...], ms_ref[...], qs_ref[...],
                gs_ref[...], bs_ref[...])
    fl = branch(feat, K_L, wlt_ref[...], ml_ref[...], ql_ref[...],
                gl_ref[...], bl_ref[...])
    cat = jnp.concatenate([fs, fl], axis=-1)
    y2 = jnp.dot(cat, wft_ref[...], preferred_element_type=jnp.float32)
    y2 = (y2 - mf_ref[...]) / qf_ref[...] * gf_ref[...] + bf_ref[...]
    out_ref[...] = _lrelu(y2)


def _edge(xt_flat, xg, c, co, wst, wlt, wft, bns, bnl, bnf):
    cp = xt_flat.shape[-1]
    full = lambda shape: pl.BlockSpec(shape, lambda i: (0,) * len(shape))
    return pl.pallas_call(
        functools.partial(_edge_body, c, co),
        grid=(N // RBE,),
        in_specs=[
            pl.BlockSpec((RBE, cp), lambda i: (i, 0)),
            pl.BlockSpec((RBE, K_L, cp), lambda i: (i, 0, 0)),
            full((2 * c, co)), full((2 * c, co)), full((2 * co, co)),
            full((1, co)), full((1, co)), full((1, co)), full((1, co)),
            full((1, co)), full((1, co)), full((1, co)), full((1, co)),
            full((1, co)), full((1, co)), full((1, co)), full((1, co)),
        ],
        out_specs=pl.BlockSpec((RBE, co), lambda i: (i, 0)),
        out_shape=jax.ShapeDtypeStruct((N, co), jnp.float32),
        interpret=_INTERPRET,
    )(xt_flat, xg, wst, wlt, wft, *bns, *bnl, *bnf)


def _head_body(cat_ref, w1t_ref, m1_ref, q1_ref, g1_ref, b1_ref,
               w2t_ref, m2_ref, q2_ref, g2_ref, b2_ref, w3t_ref, b3_ref,
               out_ref):
    h = jnp.dot(cat_ref[...], w1t_ref[...], preferred_element_type=jnp.float32)
    h = _lrelu((h - m1_ref[...]) / q1_ref[...] * g1_ref[...] + b1_ref[...])
    h = jnp.dot(h, w2t_ref[...], preferred_element_type=jnp.float32)
    h = _lrelu((h - m2_ref[...]) / q2_ref[...] * g2_ref[...] + b2_ref[...])
    out_ref[...] = (jnp.dot(h, w3t_ref[...], preferred_element_type=jnp.float32)
                    + b3_ref[...])


def _head(cat, w1t, bn1, w2t, bn2, w3t, b3):
    full = lambda shape: pl.BlockSpec(shape, lambda i: (0,) * len(shape))
    return pl.pallas_call(
        _head_body,
        grid=(BN // RB,),
        in_specs=[pl.BlockSpec((RB, 512), lambda i: (i, 0)),
                  full((512, 256)),
                  full((1, 256)), full((1, 256)), full((1, 256)),
                  full((1, 256)),
                  full((256, 128)),
                  full((1, 128)), full((1, 128)), full((1, 128)),
                  full((1, 128)),
                  full((128, 13)), full((1, 13))],
        out_specs=pl.BlockSpec((RB, 13), lambda i: (i, 0)),
        out_shape=jax.ShapeDtypeStruct((BN, 13), jnp.float32),
        interpret=_INTERPRET,
    )(cat, w1t, *bn1, w2t, *bn2, w3t, b3)


def _layer(xt, p, cout):
    c = xt.shape[-1]
    xT = jnp.transpose(xt, (0, 2, 1))
    cp = max(16, c)
    xt_p = xt if cp == c else jnp.pad(xt, ((0, 0), (0, 0), (0, cp - c)))
    x_tab = xt_p.reshape(BN, cp)
    bns, bnl, bnf = _bn_args(p['bns']), _bn_args(p['bnl']), _bn_args(p['bnf'])
    outs = []
    for b in range(B):
        i40 = _prep(xt[b], xT[b], b * N)
        xg = _gather(i40, x_tab).reshape(N, K_L, cp)
        outs.append(_edge(xt_p[b], xg, c, cout,
                          p['ws'].T, p['wl'].T, p['wf'].T, bns, bnl, bnf))
    return jnp.stack(outs)


def kernel(x, params):
    h1 = _layer(x, params['ec1'], 64)
    h2 = _layer(h1, params['ec2'], 64)
    h3 = _layer(h2, params['ec3'], 128)
    h4 = _layer(h3, params['ec4'], 256)

    cat = jnp.concatenate([h.reshape(BN, -1) for h in (h1, h2, h3, h4)],
                          axis=-1)
    out = _head(cat, params['w1'].T, _bn_args(params['bn1']),
                params['w2'].T, _bn_args(params['bn2']),
                params['w3'].T, params['b3'][None, :])
    return out.reshape(B, N, 13)
